# baseline (speedup 1.0000x reference)
import functools

import jax
import jax.numpy as jnp
from jax.experimental import pallas as pl

_HIGH = jax.lax.Precision.HIGHEST

_B = 1024
_K_RANK = 308
_MAXM = 4096
_BM = 128


def _cnn_body(img_ref, w1_ref, b1_ref, w2_ref, b2_ref, o_ref):
    bm = _BM
    img = img_ref[...].reshape(bm, 28, 28)

    slab1 = jnp.concatenate(
        [img[:, dy:dy + 24, :].reshape(bm * 24, 28) for dy in range(5)],
        axis=1)
    acc1 = jax.lax.dot(slab1, w1_ref[...].reshape(140, 480))
    z1 = jnp.maximum(acc1 + b1_ref[...], 0.0)
    zx = jnp.maximum(z1[:, :240], z1[:, 240:])
    z3 = zx.reshape(bm, 12, 2, 240)
    c1 = jnp.max(z3, axis=2)

    slab2 = jnp.concatenate(
        [c1[:, dy:dy + 8, :].reshape(bm * 8, 240) for dy in range(5)],
        axis=1)
    acc2 = jax.lax.dot(slab2, w2_ref[...].reshape(1200, 400))
    z2 = jnp.maximum(acc2 + b2_ref[...], 0.0)
    z2x = jnp.maximum(z2[:, :200], z2[:, 200:])
    z4 = z2x.reshape(bm, 4, 2, 200)
    o_ref[...] = jnp.max(z4, axis=2)


def _cnn(img2d, w1b, b1v, w2b, b2v):
    grid = _B // _BM
    return pl.pallas_call(
        _cnn_body,
        grid=(grid,),
        in_specs=[
            pl.BlockSpec((_BM * 28, 28), lambda i: (i, 0)),
            pl.BlockSpec((5, 28, 480), lambda i: (0, 0, 0)),
            pl.BlockSpec((1, 480), lambda i: (0, 0)),
            pl.BlockSpec((5, 240, 400), lambda i: (0, 0, 0)),
            pl.BlockSpec((1, 400), lambda i: (0, 0)),
        ],
        out_specs=pl.BlockSpec((_BM, 4, 200), lambda i: (i, 0, 0)),
        out_shape=jax.ShapeDtypeStruct((_B, 4, 200), jnp.float32),
    )(img2d, w1b, b1v, w2b, b2v)


def _grid_val(m):
    return jnp.float32(1.7) + jnp.float32(0.1) * m.astype(jnp.float32)


def _mega_body(h800_ref, fcw_ref, fcb_ref, gw_ref, gb_ref, o_ref):
    h = jnp.maximum(
        jax.lax.dot(h800_ref[...], fcw_ref[...]) + fcb_ref[...], 0.0)
    g = jax.lax.dot_general(h, h, (((1,), (1,)), ((), ())),
                            precision=_HIGH)
    rows = jax.lax.broadcasted_iota(jnp.int32, (_B, _B), 0)
    cols = jax.lax.broadcasted_iota(jnp.int32, (_B, _B), 1)
    eye = rows == cols
    sq = jnp.sum(jnp.where(eye, g, 0.0), axis=1)
    d2 = jnp.maximum(sq[:, None] + sq[None, :] - 2.0 * g, 0.0)
    dist = jnp.sqrt(d2)
    dist = jnp.where(eye, jnp.inf, dist)

    def bs_step(_, carry):
        lo, hi = carry
        mid = (lo + hi) // 2
        cnt = jnp.sum((dist < _grid_val(mid)).astype(jnp.int32))
        ge = cnt >= _K_RANK
        return (jnp.where(ge, lo, mid + 1), jnp.where(ge, mid, hi))

    lo0 = jnp.int32(0)
    hi0 = jnp.int32(_MAXM)
    _, m_star = jax.lax.fori_loop(0, 13, bs_step, (lo0, hi0))
    thr = _grid_val(m_star)

    ahat = jnp.where(eye, 1.0, jnp.where(dist < thr, 1.0, 0.0))
    deg = jnp.sum(ahat, axis=0)
    dinv = jax.lax.rsqrt(deg)
    xw = jax.lax.dot(h, gw_ref[...])
    s = xw * dinv[:, None]
    agg = jax.lax.dot(ahat, s, precision=_HIGH)
    out = agg * dinv[:, None] + gb_ref[...]

    mx = jnp.max(out, axis=1, keepdims=True)
    e = jnp.exp(out - mx)
    lse = jnp.log(jnp.sum(e, axis=1, keepdims=True))
    o_ref[...] = out - mx - lse


def _mega(h800, fc_w, fc_b, gcn_w, gcn_b):
    return pl.pallas_call(
        _mega_body,
        out_shape=jax.ShapeDtypeStruct((_B, 3), jnp.float32),
    )(h800, fc_w, fc_b.reshape(1, -1), gcn_w, gcn_b.reshape(1, -1))


def _banded_weights(conv1_w, conv1_b, conv2_w, conv2_b, fc_w):
    w1r = conv1_w.reshape(20, 5, 5)
    ox1 = jnp.concatenate([2 * jnp.arange(12), 2 * jnp.arange(12) + 1])
    dx1 = jnp.arange(28)[:, None, None] - ox1[None, :, None]
    s1 = (dx1 == jnp.arange(5)[None, None, :]).astype(jnp.float32)
    w1b = jnp.einsum('xjd,kyd->yxjk', s1, w1r).reshape(5, 28, 480)
    b1v = jnp.tile(conv1_b, 24).reshape(1, 480)

    ox2 = jnp.concatenate([2 * jnp.arange(4), 2 * jnp.arange(4) + 1])
    dx2 = jnp.arange(12)[:, None, None] - ox2[None, :, None]
    s2 = (dx2 == jnp.arange(5)[None, None, :]).astype(jnp.float32)
    w2b = jnp.einsum('xjd,kcyd->yxcjk', s2, conv2_w).reshape(5, 240, 400)
    b2v = jnp.tile(conv2_b, 8).reshape(1, 400)

    oy = jnp.arange(4)[:, None, None]
    ox = jnp.arange(4)[None, :, None]
    co = jnp.arange(50)[None, None, :]
    src = (co * 16 + oy * 4 + ox).reshape(800)
    fc_w_perm = fc_w[src]
    return w1b, b1v, w2b, b2v, fc_w_perm


@jax.jit
def kernel(x, conv1_w, conv1_b, conv2_w, conv2_b, fc_w, fc_b, gcn_w, gcn_b):
    img2d = x[0][:, 0].reshape(_B * 28, 28)
    w1b, b1v, w2b, b2v, fc_w_perm = _banded_weights(
        conv1_w, conv1_b, conv2_w, conv2_b, fc_w)
    c2 = _cnn(img2d, w1b, b1v, w2b, b2v)
    h800 = c2.reshape(_B, 800)
    return _mega(h800, fc_w_perm, fc_b, gcn_w, gcn_b)
